# baseline (speedup 1.0000x reference)
import functools

import jax
import jax.numpy as jnp
from jax import lax
from jax.experimental import pallas as pl
from jax.experimental.pallas import tpu as pltpu
from jax.experimental.pallas import tpu_sc as plsc

N = 10000
E = 320000
D = 128

NC = 2
NS = 16
NW = NC * NS
EPT = E // NW
B = 96
NFULL = EPT // B
REM = EPT - NFULL * B
NP = 10240
SLICE = NP // NS
ROWS_PT = NP // NS
ZR = 128

_f32 = jnp.float32

@functools.lru_cache(maxsize=None)
def _sc_mesh():
    return plsc.VectorSubcoreMesh(
        core_axis_name="c", subcore_axis_name="s",
        num_cores=NC, num_subcores=NS)


def _deg_body(row_hbm, deg_out, idx_v, hist_v, tmp_v, acc_v, hists_sp):
    c = lax.axis_index("c")
    s = lax.axis_index("s")
    wid = c * NS + s

    def zero_body(i, _):
        hist_v[pl.ds(i * 16, 16)] = jnp.zeros((16,), _f32)
        return _
    lax.fori_loop(0, NP // 16, zero_body, None)

    pltpu.sync_copy(row_hbm.at[pl.ds(pl.multiple_of(wid * EPT, 16), EPT)], idx_v)

    ones = jnp.ones((16,), _f32)

    def hist_body(i, _):
        idx = idx_v[pl.ds(i * 16, 16)]
        plsc.addupdate_scatter(hist_v, [idx], ones)
        return _
    lax.fori_loop(0, EPT // 16, hist_body, None)

    pltpu.sync_copy(hist_v, hists_sp.at[s])
    plsc.subcore_barrier()

    base = pl.multiple_of(s * SLICE, 16)
    pltpu.sync_copy(hists_sp.at[0, pl.ds(base, SLICE)], acc_v)

    def merge_body(t, _):
        pltpu.sync_copy(hists_sp.at[t, pl.ds(base, SLICE)], tmp_v)

        def add_body(k, _2):
            sl = pl.ds(k * 16, 16)
            acc_v[sl] = acc_v[sl] + tmp_v[sl]
            return _2
        lax.fori_loop(0, SLICE // 16, add_body, None)
        return _
    lax.fori_loop(1, NS, merge_body, None)

    pltpu.sync_copy(acc_v, deg_out.at[c, pl.ds(base, SLICE)])


def _deg_call(row):
    return pl.kernel(
        _deg_body,
        out_type=jax.ShapeDtypeStruct((NC, NP), _f32),
        mesh=_sc_mesh(),
        compiler_params=pltpu.CompilerParams(needs_layout_passes=False),
        scratch_types=[
            pltpu.VMEM((EPT,), jnp.int32),
            pltpu.VMEM((NP,), _f32),
            pltpu.VMEM((SLICE,), _f32),
            pltpu.VMEM((SLICE,), _f32),
            pltpu.VMEM_SHARED((NS, NP), _f32),
        ],
    )(row)


def _edge_body(xs_hbm, row_hbm, col_hbm, acc_out,
               rflat_v, cidx_v, col_rem, buf0, buf1,
               acc_sp, isem, sem0, sem1):
    c = lax.axis_index("c")
    s = lax.axis_index("s")
    wid = c * NS + s
    ebase = pl.multiple_of(wid * EPT, 16)
    nbase = s * ROWS_PT

    pltpu.async_copy(row_hbm.at[pl.ds(ebase, EPT)], rflat_v, isem)

    def fire(j, _):
        off = pl.multiple_of(ebase + j * B, 8)
        pltpu.async_copy(col_hbm.at[pl.ds(off, B)], cidx_v.at[j], isem)
        return _
    lax.fori_loop(0, NFULL, fire, None)
    pltpu.async_copy(col_hbm.at[pl.ds(ebase + NFULL * B, REM)], col_rem, isem)

    def zrow(i, _):
        buf0[i // (D // 16), pl.ds((i % (D // 16)) * 16, 16)] = (
            jnp.zeros((16,), _f32))
        return _
    lax.fori_loop(0, 64 * (D // 16), zrow, None)

    def zcp(k, _):
        pltpu.sync_copy(buf0.at[pl.ds(0, 64)],
                        acc_sp.at[pl.ds(nbase + k * 64, 64)])
        return _
    lax.fori_loop(0, ROWS_PT // 64, zcp, None)

    pltpu.make_async_copy(row_hbm.at[pl.ds(ebase, EPT)], rflat_v, isem).wait()

    def drain(j, _):
        pltpu.make_async_copy(col_hbm.at[pl.ds(ebase, B)], cidx_v.at[j],
                              isem).wait()
        return _
    lax.fori_loop(0, NFULL, drain, None)
    pltpu.make_async_copy(col_hbm.at[pl.ds(ebase, REM)], col_rem, isem).wait()

    def gidx(j):
        return rflat_v.at[pl.ds(j * B, B)]

    pltpu.async_copy(xs_hbm.at[gidx(0)], buf0, sem0)
    plsc.subcore_barrier()

    def pair(i, _):
        j0 = 2 * i
        pltpu.async_copy(xs_hbm.at[gidx(j0 + 1)], buf1, sem1)
        pltpu.make_async_copy(xs_hbm.at[gidx(j0)], buf0, sem0).wait()
        pltpu.sync_copy(buf0, acc_sp.at[cidx_v.at[j0]], add=True)

        @pl.when(j0 + 2 < NFULL)
        def _():
            pltpu.async_copy(xs_hbm.at[gidx(j0 + 2)], buf0, sem0)
        pltpu.make_async_copy(xs_hbm.at[gidx(j0 + 1)], buf1, sem1).wait()
        pltpu.sync_copy(buf1, acc_sp.at[cidx_v.at[j0 + 1]], add=True)
        return _
    lax.fori_loop(0, NFULL // 2, pair, None)

    pltpu.async_copy(
        xs_hbm.at[rflat_v.at[pl.ds(NFULL * B, REM)]],
        buf1.at[pl.ds(0, REM)], sem0).wait()
    pltpu.sync_copy(buf1.at[pl.ds(0, REM)], acc_sp.at[col_rem], add=True)

    plsc.subcore_barrier()

    sl = pl.ds(nbase, ROWS_PT)
    pltpu.sync_copy(acc_sp.at[sl], acc_out.at[c, sl])


def _edge_call(xs, row, col):
    return pl.kernel(
        _edge_body,
        out_type=jax.ShapeDtypeStruct((NC, NP, D), _f32),
        mesh=_sc_mesh(),
        scratch_types=[
            pltpu.VMEM((EPT,), jnp.int32),
            pltpu.VMEM((NFULL, B), jnp.int32),
            pltpu.VMEM((REM,), jnp.int32),
            pltpu.VMEM((B, D), _f32),
            pltpu.VMEM((B, D), _f32),
            pltpu.VMEM_SHARED((NP, D), _f32),
            pltpu.SemaphoreType.DMA,
            pltpu.SemaphoreType.DMA,
            pltpu.SemaphoreType.DMA,
        ],
    )(xs, row, col)


BN = 10000


def _mmscale_body(nf_ref, w_ref, deg_ref, x_ref, xs_ref):
    x = jnp.dot(nf_ref[...], w_ref[...], preferred_element_type=_f32)
    x_ref[...] = x
    deg = deg_ref[0] + deg_ref[1]
    dis = jnp.where(deg > 0.0, lax.rsqrt(deg), 0.0)
    xs_ref[...] = x * dis


def _mmscale_call(nf, w, deg3):
    return pl.pallas_call(
        _mmscale_body,
        grid=(N // BN,),
        in_specs=[
            pl.BlockSpec((BN, D), lambda i: (i, 0)),
            pl.BlockSpec((D, D), lambda i: (0, 0)),
            pl.BlockSpec((2, BN, 1), lambda i: (0, i, 0)),
        ],
        out_specs=[
            pl.BlockSpec((BN, D), lambda i: (i, 0)),
            pl.BlockSpec((BN, D), lambda i: (i, 0)),
        ],
        out_shape=[
            jax.ShapeDtypeStruct((N, D), _f32),
            jax.ShapeDtypeStruct((N, D), _f32),
        ],
    )(nf, w, deg3)


def _out_body(acc_ref, deg_ref, x_ref, o_ref):
    a = acc_ref[0] + acc_ref[1]
    deg = deg_ref[0] + deg_ref[1]
    dis = jnp.where(deg > 0.0, lax.rsqrt(deg), 0.0)
    o_ref[...] = a * dis + x_ref[...]


def _out_call(acc_parts, deg3, x):
    return pl.pallas_call(
        _out_body,
        grid=(N // BN,),
        in_specs=[
            pl.BlockSpec((2, BN, D), lambda i: (0, i, 0)),
            pl.BlockSpec((2, BN, 1), lambda i: (0, i, 0)),
            pl.BlockSpec((BN, D), lambda i: (i, 0)),
        ],
        out_specs=pl.BlockSpec((BN, D), lambda i: (i, 0)),
        out_shape=jax.ShapeDtypeStruct((N, D), _f32),
    )(acc_parts, deg3, x)


def kernel(node_feature, edge_index, lin_weight):
    row = edge_index[0]
    col = edge_index[1]
    deg_parts = _deg_call(row)
    deg3 = deg_parts.reshape(NC, NP, 1)
    x, xs = _mmscale_call(node_feature, lin_weight, deg3)
    acc_parts = _edge_call(xs, row, col)
    out = _out_call(acc_parts, deg3, x)
    return out


# speedup vs baseline: 1.0567x; 1.0567x over previous
import functools

import jax
import jax.numpy as jnp
from jax import lax
from jax.experimental import pallas as pl
from jax.experimental.pallas import tpu as pltpu
from jax.experimental.pallas import tpu_sc as plsc

N = 10000
E = 320000
D = 128

NC = 2
NS = 16
NW = NC * NS
EPT = E // NW
B = 96
NFULL = EPT // B
REM = EPT - NFULL * B
NP = 10240
SLICE = NP // NS
ROWS_PT = NP // NS
ZR = 128

_f32 = jnp.float32

@functools.lru_cache(maxsize=None)
def _sc_mesh():
    return plsc.VectorSubcoreMesh(
        core_axis_name="c", subcore_axis_name="s",
        num_cores=NC, num_subcores=NS)


def _deg_body(ei_hbm, deg_out, idx_v, hist_v, tmp_v, acc_v, hists_sp):
    c = lax.axis_index("c")
    s = lax.axis_index("s")
    wid = c * NS + s

    def zero_body(i, _):
        hist_v[pl.ds(i * 16, 16)] = jnp.zeros((16,), _f32)
        return _
    lax.fori_loop(0, NP // 16, zero_body, None)

    pltpu.sync_copy(
        ei_hbm.at[pl.ds(pl.multiple_of(wid * EPT, 16), EPT)], idx_v)

    ones = jnp.ones((16,), _f32)

    def hist_body(i, _):
        idx = idx_v[pl.ds(i * 16, 16)]
        plsc.addupdate_scatter(hist_v, [idx], ones)
        return _
    lax.fori_loop(0, EPT // 16, hist_body, None)

    pltpu.sync_copy(hist_v, hists_sp.at[s])
    plsc.subcore_barrier()

    base = pl.multiple_of(s * SLICE, 16)
    pltpu.sync_copy(hists_sp.at[0, pl.ds(base, SLICE)], acc_v)

    def merge_body(t, _):
        pltpu.sync_copy(hists_sp.at[t, pl.ds(base, SLICE)], tmp_v)

        def add_body(k, _2):
            sl = pl.ds(k * 16, 16)
            acc_v[sl] = acc_v[sl] + tmp_v[sl]
            return _2
        lax.fori_loop(0, SLICE // 16, add_body, None)
        return _
    lax.fori_loop(1, NS, merge_body, None)

    pltpu.sync_copy(acc_v, deg_out.at[c, pl.ds(base, SLICE)])


def _deg_call(edge_index):
    return pl.kernel(
        _deg_body,
        out_type=jax.ShapeDtypeStruct((NC, NP), _f32),
        mesh=_sc_mesh(),
        compiler_params=pltpu.CompilerParams(needs_layout_passes=False),
        scratch_types=[
            pltpu.VMEM((EPT,), jnp.int32),
            pltpu.VMEM((NP,), _f32),
            pltpu.VMEM((SLICE,), _f32),
            pltpu.VMEM((SLICE,), _f32),
            pltpu.VMEM_SHARED((NS, NP), _f32),
        ],
    )(edge_index)


def _edge_body(xs_hbm, ei_hbm, acc_out,
               rflat_v, cidx_v, col_rem, buf0, buf1,
               acc_sp, isem, sem0, sem1):
    c = lax.axis_index("c")
    s = lax.axis_index("s")
    wid = c * NS + s
    ebase = pl.multiple_of(wid * EPT, 16)
    nbase = s * ROWS_PT

    pltpu.async_copy(ei_hbm.at[pl.ds(ebase, EPT)], rflat_v, isem)

    def fire(j, _):
        off = pl.multiple_of(E + ebase + j * B, 8)
        pltpu.async_copy(ei_hbm.at[pl.ds(off, B)], cidx_v.at[j], isem)
        return _
    lax.fori_loop(0, NFULL, fire, None)
    pltpu.async_copy(
        ei_hbm.at[pl.ds(E + ebase + NFULL * B, REM)], col_rem, isem)

    def zrow(i, _):
        buf0[i // (D // 16), pl.ds((i % (D // 16)) * 16, 16)] = (
            jnp.zeros((16,), _f32))
        return _
    lax.fori_loop(0, 64 * (D // 16), zrow, None)

    def zcp(k, _):
        pltpu.sync_copy(buf0.at[pl.ds(0, 64)],
                        acc_sp.at[pl.ds(nbase + k * 64, 64)])
        return _
    lax.fori_loop(0, ROWS_PT // 64, zcp, None)

    pltpu.make_async_copy(
        ei_hbm.at[pl.ds(ebase, EPT)], rflat_v, isem).wait()

    def drain(j, _):
        pltpu.make_async_copy(ei_hbm.at[pl.ds(ebase, B)], cidx_v.at[j],
                              isem).wait()
        return _
    lax.fori_loop(0, NFULL, drain, None)
    pltpu.make_async_copy(
        ei_hbm.at[pl.ds(ebase, REM)], col_rem, isem).wait()

    def gidx(j):
        return rflat_v.at[pl.ds(j * B, B)]

    pltpu.async_copy(xs_hbm.at[gidx(0)], buf0, sem0)
    plsc.subcore_barrier()

    def pair(i, _):
        j0 = 2 * i
        pltpu.async_copy(xs_hbm.at[gidx(j0 + 1)], buf1, sem1)
        pltpu.make_async_copy(xs_hbm.at[gidx(j0)], buf0, sem0).wait()
        pltpu.sync_copy(buf0, acc_sp.at[cidx_v.at[j0]], add=True)

        @pl.when(j0 + 2 < NFULL)
        def _():
            pltpu.async_copy(xs_hbm.at[gidx(j0 + 2)], buf0, sem0)
        pltpu.make_async_copy(xs_hbm.at[gidx(j0 + 1)], buf1, sem1).wait()
        pltpu.sync_copy(buf1, acc_sp.at[cidx_v.at[j0 + 1]], add=True)
        return _
    lax.fori_loop(0, NFULL // 2, pair, None)

    pltpu.async_copy(
        xs_hbm.at[rflat_v.at[pl.ds(NFULL * B, REM)]],
        buf1.at[pl.ds(0, REM)], sem0).wait()
    pltpu.sync_copy(buf1.at[pl.ds(0, REM)], acc_sp.at[col_rem], add=True)

    plsc.subcore_barrier()

    sl = pl.ds(nbase, ROWS_PT)
    pltpu.sync_copy(acc_sp.at[sl], acc_out.at[c, sl])


def _edge_call(xs, edge_index):
    return pl.kernel(
        _edge_body,
        out_type=jax.ShapeDtypeStruct((NC, NP, D), _f32),
        mesh=_sc_mesh(),
        scratch_types=[
            pltpu.VMEM((EPT,), jnp.int32),
            pltpu.VMEM((NFULL, B), jnp.int32),
            pltpu.VMEM((REM,), jnp.int32),
            pltpu.VMEM((B, D), _f32),
            pltpu.VMEM((B, D), _f32),
            pltpu.VMEM_SHARED((NP, D), _f32),
            pltpu.SemaphoreType.DMA,
            pltpu.SemaphoreType.DMA,
            pltpu.SemaphoreType.DMA,
        ],
    )(xs, edge_index)


BN = 10000


def _mmscale_body(nf_ref, w_ref, deg_ref, x_ref, xs_ref):
    x = jnp.dot(nf_ref[...], w_ref[...], preferred_element_type=_f32)
    x_ref[...] = x
    deg = deg_ref[0] + deg_ref[1]
    dis = jnp.where(deg > 0.0, lax.rsqrt(deg), 0.0)
    xs_ref[...] = x * dis


def _mmscale_call(nf, w, deg3):
    return pl.pallas_call(
        _mmscale_body,
        grid=(N // BN,),
        in_specs=[
            pl.BlockSpec((BN, D), lambda i: (i, 0)),
            pl.BlockSpec((D, D), lambda i: (0, 0)),
            pl.BlockSpec((2, BN, 1), lambda i: (0, i, 0)),
        ],
        out_specs=[
            pl.BlockSpec((BN, D), lambda i: (i, 0)),
            pl.BlockSpec((BN, D), lambda i: (i, 0)),
        ],
        out_shape=[
            jax.ShapeDtypeStruct((N, D), _f32),
            jax.ShapeDtypeStruct((N, D), _f32),
        ],
    )(nf, w, deg3)


def _out_body(acc_ref, deg_ref, x_ref, o_ref):
    a = acc_ref[0] + acc_ref[1]
    deg = deg_ref[0] + deg_ref[1]
    dis = jnp.where(deg > 0.0, lax.rsqrt(deg), 0.0)
    o_ref[...] = a * dis + x_ref[...]


def _out_call(acc_parts, deg3, x):
    return pl.pallas_call(
        _out_body,
        grid=(N // BN,),
        in_specs=[
            pl.BlockSpec((2, BN, D), lambda i: (0, i, 0)),
            pl.BlockSpec((2, BN, 1), lambda i: (0, i, 0)),
            pl.BlockSpec((BN, D), lambda i: (i, 0)),
        ],
        out_specs=pl.BlockSpec((BN, D), lambda i: (i, 0)),
        out_shape=jax.ShapeDtypeStruct((N, D), _f32),
    )(acc_parts, deg3, x)


def kernel(node_feature, edge_index, lin_weight):
    ei_flat = edge_index.reshape(-1)
    deg_parts = _deg_call(ei_flat)
    deg3 = deg_parts.reshape(NC, NP, 1)
    x, xs = _mmscale_call(node_feature, lin_weight, deg3)
    acc_parts = _edge_call(xs, ei_flat)
    out = _out_call(acc_parts, deg3, x)
    return out


# speedup vs baseline: 1.1384x; 1.0773x over previous
import functools

import jax
import jax.numpy as jnp
from jax import lax
from jax.experimental import pallas as pl
from jax.experimental.pallas import tpu as pltpu
from jax.experimental.pallas import tpu_sc as plsc

N = 10000
E = 320000
D = 128

NC = 2
NS = 16
NW = NC * NS
EPT = E // NW
B = 96
NFULL = EPT // B
REM = EPT - NFULL * B
NP = 10240
SLICE = NP // NS
ROWS_PT = NP // NS
ZR = 128

_f32 = jnp.float32

@functools.lru_cache(maxsize=None)
def _sc_mesh():
    return plsc.VectorSubcoreMesh(
        core_axis_name="c", subcore_axis_name="s",
        num_cores=NC, num_subcores=NS)


def _deg_body(ei_hbm, deg_out, idx_v, hist_v, tmp_v, acc_v, hists_sp):
    c = lax.axis_index("c")
    s = lax.axis_index("s")
    wid = c * NS + s

    def zero_body(i, _):
        hist_v[pl.ds(i * 16, 16)] = jnp.zeros((16,), _f32)
        return _
    lax.fori_loop(0, NP // 16, zero_body, None)

    pltpu.sync_copy(
        ei_hbm.at[pl.ds(pl.multiple_of(wid * EPT, 16), EPT)], idx_v)

    ones = jnp.ones((16,), _f32)

    def hist_body(i, _):
        idx = idx_v[pl.ds(i * 16, 16)]
        plsc.addupdate_scatter(hist_v, [idx], ones)
        return _
    lax.fori_loop(0, EPT // 16, hist_body, None)

    pltpu.sync_copy(hist_v, hists_sp.at[s])
    plsc.subcore_barrier()

    base = pl.multiple_of(s * SLICE, 16)
    pltpu.sync_copy(hists_sp.at[0, pl.ds(base, SLICE)], acc_v)

    def merge_body(t, _):
        pltpu.sync_copy(hists_sp.at[t, pl.ds(base, SLICE)], tmp_v)

        def add_body(k, _2):
            sl = pl.ds(k * 16, 16)
            acc_v[sl] = acc_v[sl] + tmp_v[sl]
            return _2
        lax.fori_loop(0, SLICE // 16, add_body, None)
        return _
    lax.fori_loop(1, NS, merge_body, None)

    pltpu.sync_copy(acc_v, deg_out.at[c, pl.ds(base, SLICE)])


def _deg_call(edge_index):
    return pl.kernel(
        _deg_body,
        out_type=jax.ShapeDtypeStruct((NC, NP), _f32),
        mesh=_sc_mesh(),
        compiler_params=pltpu.CompilerParams(needs_layout_passes=False),
        scratch_types=[
            pltpu.VMEM((EPT,), jnp.int32),
            pltpu.VMEM((NP,), _f32),
            pltpu.VMEM((SLICE,), _f32),
            pltpu.VMEM((SLICE,), _f32),
            pltpu.VMEM_SHARED((NS, NP), _f32),
        ],
    )(edge_index)


def _edge_body(xs_hbm, ei_hbm, acc_out,
               rflat_v, cidx_v, col_rem, buf0, buf1,
               acc_sp, isem, sem0, sem1):
    c = lax.axis_index("c")
    s = lax.axis_index("s")
    wid = c * NS + s
    ebase = pl.multiple_of(wid * EPT, 16)
    nbase = s * ROWS_PT

    pltpu.async_copy(ei_hbm.at[pl.ds(ebase, EPT)], rflat_v, isem)

    def fire(j, _):
        off = pl.multiple_of(E + ebase + j * B, 8)
        pltpu.async_copy(ei_hbm.at[pl.ds(off, B)], cidx_v.at[j], isem)
        return _
    lax.fori_loop(0, NFULL, fire, None)
    pltpu.async_copy(
        ei_hbm.at[pl.ds(E + ebase + NFULL * B, REM)], col_rem, isem)

    def zrow(i, _):
        buf0[i // (D // 16), pl.ds((i % (D // 16)) * 16, 16)] = (
            jnp.zeros((16,), _f32))
        return _
    lax.fori_loop(0, 64 * (D // 16), zrow, None)

    def zcp(k, _):
        pltpu.sync_copy(buf0.at[pl.ds(0, 64)],
                        acc_sp.at[pl.ds(nbase + k * 64, 64)])
        return _
    lax.fori_loop(0, ROWS_PT // 64, zcp, None)

    pltpu.make_async_copy(
        ei_hbm.at[pl.ds(ebase, EPT)], rflat_v, isem).wait()

    def drain(j, _):
        pltpu.make_async_copy(ei_hbm.at[pl.ds(ebase, B)], cidx_v.at[j],
                              isem).wait()
        return _
    lax.fori_loop(0, NFULL, drain, None)
    pltpu.make_async_copy(
        ei_hbm.at[pl.ds(ebase, REM)], col_rem, isem).wait()

    def gidx(j):
        return rflat_v.at[pl.ds(j * B, B)]

    pltpu.async_copy(xs_hbm.at[gidx(0)], buf0, sem0)
    plsc.subcore_barrier()

    def pair(i, _):
        j0 = 2 * i
        pltpu.async_copy(xs_hbm.at[gidx(j0 + 1)], buf1, sem1)
        pltpu.make_async_copy(xs_hbm.at[gidx(j0)], buf0, sem0).wait()
        pltpu.sync_copy(buf0, acc_sp.at[cidx_v.at[j0]], add=True)

        @pl.when(j0 + 2 < NFULL)
        def _():
            pltpu.async_copy(xs_hbm.at[gidx(j0 + 2)], buf0, sem0)
        pltpu.make_async_copy(xs_hbm.at[gidx(j0 + 1)], buf1, sem1).wait()
        pltpu.sync_copy(buf1, acc_sp.at[cidx_v.at[j0 + 1]], add=True)
        return _
    lax.fori_loop(0, NFULL // 2, pair, None)

    pltpu.async_copy(
        xs_hbm.at[rflat_v.at[pl.ds(NFULL * B, REM)]],
        buf1.at[pl.ds(0, REM)], sem0).wait()
    pltpu.sync_copy(buf1.at[pl.ds(0, REM)], acc_sp.at[col_rem], add=True)

    plsc.subcore_barrier()

    sl = pl.ds(nbase, ROWS_PT)
    pltpu.sync_copy(acc_sp.at[sl], acc_out.at[c, sl])


def _edge_call(xs, edge_index):
    return pl.kernel(
        _edge_body,
        out_type=jax.ShapeDtypeStruct((NC, NP, D), _f32),
        mesh=_sc_mesh(),
        scratch_types=[
            pltpu.VMEM((EPT,), jnp.int32),
            pltpu.VMEM((NFULL, B), jnp.int32),
            pltpu.VMEM((REM,), jnp.int32),
            pltpu.VMEM((B, D), _f32),
            pltpu.VMEM((B, D), _f32),
            pltpu.VMEM_SHARED((NP, D), _f32),
            pltpu.SemaphoreType.DMA,
            pltpu.SemaphoreType.DMA,
            pltpu.SemaphoreType.DMA,
        ],
    )(xs, edge_index)


BN = 10000


def _mmscale_body(nf_ref, w_ref, deg_ref, x_ref, xs_ref):
    x = jnp.dot(nf_ref[...], w_ref[...], preferred_element_type=_f32)
    x_ref[...] = x
    deg = deg_ref[0, :BN] + deg_ref[1, :BN]
    dis = jnp.where(deg > 0.0, lax.rsqrt(deg), 0.0)
    xs_ref[...] = x * dis[:, None]


def _mmscale_call(nf, w, deg_parts):
    return pl.pallas_call(
        _mmscale_body,
        grid=(N // BN,),
        in_specs=[
            pl.BlockSpec((BN, D), lambda i: (i, 0)),
            pl.BlockSpec((D, D), lambda i: (0, 0)),
            pl.BlockSpec((2, NP), lambda i: (0, 0)),
        ],
        out_specs=[
            pl.BlockSpec((BN, D), lambda i: (i, 0)),
            pl.BlockSpec((BN, D), lambda i: (i, 0)),
        ],
        out_shape=[
            jax.ShapeDtypeStruct((N, D), _f32),
            jax.ShapeDtypeStruct((N, D), _f32),
        ],
    )(nf, w, deg_parts)


def _out_body(acc_ref, deg_ref, x_ref, o_ref):
    a = acc_ref[0] + acc_ref[1]
    deg = deg_ref[0, :BN] + deg_ref[1, :BN]
    dis = jnp.where(deg > 0.0, lax.rsqrt(deg), 0.0)
    o_ref[...] = a * dis[:, None] + x_ref[...]


def _out_call(acc_parts, deg_parts, x):
    return pl.pallas_call(
        _out_body,
        grid=(N // BN,),
        in_specs=[
            pl.BlockSpec((2, BN, D), lambda i: (0, i, 0)),
            pl.BlockSpec((2, NP), lambda i: (0, 0)),
            pl.BlockSpec((BN, D), lambda i: (i, 0)),
        ],
        out_specs=pl.BlockSpec((BN, D), lambda i: (i, 0)),
        out_shape=jax.ShapeDtypeStruct((N, D), _f32),
    )(acc_parts, deg_parts, x)


def kernel(node_feature, edge_index, lin_weight):
    ei_flat = edge_index.reshape(-1)
    deg_parts = _deg_call(ei_flat)
    x, xs = _mmscale_call(node_feature, lin_weight, deg_parts)
    acc_parts = _edge_call(xs, ei_flat)
    out = _out_call(acc_parts, deg_parts, x)
    return out
